# Initial kernel scaffold; baseline (speedup 1.0000x reference)
import jax
import jax.numpy as jnp
from jax.experimental import pallas as pl


def kernel(x, g1_fc1_w, g1_fc1_b, g1_bn1_g, g1_bn1_b, g1_rel, g1_mr_w, g1_mr_b, g1_fc2_w, g1_fc2_b, g1_bn2_g, g1_bn2_b, g2_fc1_w, g2_fc1_b, g2_bn1_g, g2_bn1_b, g2_rel, g2_mr_w, g2_mr_b, g2_fc2_w, g2_fc2_b, g2_bn2_g, g2_bn2_b, s1_gn_w, s1_gn_b, s1_sq1_w, s1_sq2_w, s1_gwc_w, s1_gwc_b, s1_pwc1_w, s1_pwc2_w, s2_gn_w, s2_gn_b, s2_sq1_w, s2_sq2_w, s2_gwc_w, s2_gwc_b, s2_pwc1_w, s2_pwc2_w):
    raise NotImplementedError("write your pallas kernel here")


# speedup vs baseline: 223.9745x; 223.9745x over previous
import functools

import jax
import jax.numpy as jnp
from jax import lax
from jax.experimental import pallas as pl
from jax.experimental.pallas import tpu as pltpu
from jax.experimental.pallas import tpu_sc as plsc

_B, _C, _H, _W = 16, 96, 32, 32
_N = _H * _W
_K = 7
_R = _B * _N
_CNT = float(_B * _N)


_PH = lax.Precision.HIGHEST


def _fc_stats_body(x_ref, w_ref, b_ref, y_ref, st_ref):
    bidx = pl.program_id(0)
    y = jnp.dot(w_ref[...], x_ref[0], preferred_element_type=jnp.float32)
    y = y + b_ref[...]
    y_ref[0] = y

    @pl.when(bidx == 0)
    def _():
        st_ref[...] = jnp.zeros_like(st_ref)

    acc = st_ref[...]
    for nt in range(_N // 128):
        acc = acc + y[:, nt * 128:(nt + 1) * 128]
    st_ref[...] = acc


def _fc_stats(x, w, b2):
    cout = w.shape[0]
    return pl.pallas_call(
        _fc_stats_body,
        grid=(_B,),
        in_specs=[
            pl.BlockSpec((1, x.shape[1], _N), lambda b: (b, 0, 0)),
            pl.BlockSpec(w.shape, lambda b: (0, 0)),
            pl.BlockSpec(b2.shape, lambda b: (0, 0)),
        ],
        out_specs=[
            pl.BlockSpec((1, cout, _N), lambda b: (b, 0, 0)),
            pl.BlockSpec((cout, 128), lambda b: (0, 0)),
        ],
        out_shape=[
            jax.ShapeDtypeStruct((_B, cout, _N), jnp.float32),
            jax.ShapeDtypeStruct((cout, 128), jnp.float32),
        ],
    )(x, w, b2)


def _center_var_body(y_ref, st_ref, v_ref):
    bidx = pl.program_id(0)
    mean = jnp.sum(st_ref[...], axis=1, keepdims=True) / _CNT
    d = y_ref[0] - mean
    dd = d * d

    @pl.when(bidx == 0)
    def _():
        v_ref[...] = jnp.zeros_like(v_ref)

    acc = v_ref[...]
    for nt in range(_N // 128):
        acc = acc + dd[:, nt * 128:(nt + 1) * 128]
    v_ref[...] = acc


def _center_var(y, st):
    return pl.pallas_call(
        _center_var_body,
        grid=(_B,),
        in_specs=[
            pl.BlockSpec((1, _C, _N), lambda b: (b, 0, 0)),
            pl.BlockSpec((_C, 128), lambda b: (0, 0)),
        ],
        out_specs=pl.BlockSpec((_C, 128), lambda b: (0, 0)),
        out_shape=jax.ShapeDtypeStruct((_C, 128), jnp.float32),
    )(y, st)


def _knn_body(yr_ref, st_ref, v_ref, g_ref, b_ref, rel_ref, ycn_ref, ync_ref,
              idx_ref):
    bidx = pl.program_id(0)
    mean = jnp.sum(st_ref[...], axis=1, keepdims=True) / _CNT
    var = jnp.sum(v_ref[...], axis=1, keepdims=True) / _CNT
    y = (yr_ref[0] - mean) * (1.0 / jnp.sqrt(var + 1e-5)) * g_ref[...] \
        + b_ref[...]
    ycn_ref[0] = y
    yt = jnp.transpose(y)
    ync_ref[0] = jnp.concatenate(
        [yt, jnp.zeros((_N, 128 - _C), jnp.float32)], axis=1)
    nrm = jnp.sqrt(jnp.sum(y * y, axis=0, keepdims=True))
    xn = y * (1.0 / jnp.maximum(nrm, 1e-12))
    xnt = jnp.transpose(xn)
    sqr = jnp.sum(xn * xn, axis=0, keepdims=True)
    sqc = jnp.transpose(sqr)

    ch = 128
    imin = jnp.int32(-2**31)
    vals = []
    gidx = []
    lio = lax.broadcasted_iota(jnp.int32, (_N, ch), 1)
    for c in range(_N // ch):
        sl = slice(c * ch, (c + 1) * ch)
        inner = jnp.dot(xnt, xn[:, sl], preferred_element_type=jnp.float32)
        neg = -(((sqc - 2.0 * inner) + sqr[:, sl]) + rel_ref[0, :, sl])
        bits = lax.bitcast_convert_type(neg, jnp.int32)
        qk = jnp.where(bits >= 0, bits, (imin - bits) - 1)
        for _ in range(_K):
            rmax = jnp.max(qk, axis=1, keepdims=True)
            cand = jnp.where(qk == rmax, lio, ch)
            argl = jnp.min(cand, axis=1, keepdims=True)
            vals.append(rmax)
            gidx.append(argl + c * ch)
            qk = jnp.where(lio == argl, imin, qk)
    v = jnp.concatenate(vals, axis=1)
    g = jnp.concatenate(gidx, axis=1)
    cols = []
    for _ in range(_K):
        rmax = jnp.max(v, axis=1, keepdims=True)
        cand = jnp.where(v == rmax, g, _N)
        chosen = jnp.min(cand, axis=1, keepdims=True)
        cols.append(chosen)
        v = jnp.where(g == chosen, imin, v)
    idx = jnp.concatenate(cols + [jnp.zeros((_N, 1), jnp.int32)], axis=1)
    idx_ref[0] = idx + bidx * _N


def _knn(yr, st, vv, g2, b2, rel):
    return pl.pallas_call(
        _knn_body,
        grid=(_B,),
        in_specs=[
            pl.BlockSpec((1, _C, _N), lambda b: (b, 0, 0)),
            pl.BlockSpec((_C, 128), lambda b: (0, 0)),
            pl.BlockSpec((_C, 128), lambda b: (0, 0)),
            pl.BlockSpec((_C, 1), lambda b: (0, 0)),
            pl.BlockSpec((_C, 1), lambda b: (0, 0)),
            pl.BlockSpec((1, _N, _N), lambda b: (0, 0, 0)),
        ],
        out_specs=[
            pl.BlockSpec((1, _C, _N), lambda b: (b, 0, 0)),
            pl.BlockSpec((1, _N, 128), lambda b: (b, 0, 0)),
            pl.BlockSpec((1, _N, 8), lambda b: (b, 0, 0)),
        ],
        out_shape=[
            jax.ShapeDtypeStruct((_B, _C, _N), jnp.float32),
            jax.ShapeDtypeStruct((_B, _N, 128), jnp.float32),
            jax.ShapeDtypeStruct((_B, _N, 8), jnp.int32),
        ],
    )(yr, st, vv, g2, b2, rel)


def _gather_max(table, idx2d):
    info = plsc.get_sparse_core_info()
    nw = info.num_cores * info.num_subcores
    nodes_per_w = _R // nw
    chn = 64
    n_chunks = nodes_per_w // chn
    mesh = plsc.VectorSubcoreMesh(core_axis_name="c", subcore_axis_name="s")
    scratch = (
        [pltpu.VMEM((chn,), jnp.int32) for _ in range(_K)]
        + [pltpu.VMEM((chn, 128), jnp.float32) for _ in range(_K)]
        + [pltpu.VMEM((chn, _C), jnp.float32), pltpu.SemaphoreType.DMA]
    )

    @functools.partial(
        pl.kernel,
        mesh=mesh,
        out_type=jax.ShapeDtypeStruct((_R, _C), jnp.float32),
        scratch_types=scratch,
    )
    def k(table_hbm, idx_hbm, out_hbm, *refs):
        idxb = refs[0:_K]
        rows = refs[_K:2 * _K]
        outv = refs[2 * _K]
        sem = refs[2 * _K + 1]
        wid = lax.axis_index("s") * info.num_cores + lax.axis_index("c")

        def chunk_body(ci, carry):
            nb = wid * nodes_per_w + ci * chn
            nrow = nb // 128
            ncol = nb % 128
            for j in range(_K):
                pltpu.sync_copy(idx_hbm.at[j * 128 + nrow, pl.ds(ncol, chn)],
                                idxb[j])
            copies = [
                pltpu.async_copy(table_hbm.at[idxb[j]], rows[j], sem)
                for j in range(_K)
            ]
            for cp in copies:
                cp.wait()

            def node_body(n, c2):
                for cc in range(_C // 16):
                    sl = pl.ds(cc * 16, 16)
                    m = rows[0][n, sl]
                    for j in range(1, _K):
                        m = jnp.maximum(m, rows[j][n, sl])
                    outv[n, sl] = m
                return c2

            lax.fori_loop(0, chn, node_body, 0)
            pltpu.sync_copy(outv, out_hbm.at[pl.ds(nb, chn)])
            return carry

        lax.fori_loop(0, n_chunks, chunk_body, 0)

    return k(table, idx2d)


def _mr_fc2_body(ycn_ref, mx_ref, wy_ref, wd_ref, mb_ref, w2_ref, b2_ref,
                 z_ref, st_ref):
    bidx = pl.program_id(0)
    y = ycn_ref[0]
    d = jnp.transpose(mx_ref[0]) - y
    parts = []
    for g in range(4):
        so = slice(g * 48, (g + 1) * 48)
        si = slice(g * 24, (g + 1) * 24)
        pg = jnp.dot(wy_ref[...][so], y[si], preferred_element_type=jnp.float32)
        pg = pg + jnp.dot(wd_ref[...][so], d[si],
                          preferred_element_type=jnp.float32)
        parts.append(pg)
    m = jnp.concatenate(parts, axis=0) + mb_ref[...]
    mu = jnp.mean(m, axis=1, keepdims=True)
    mc = m - mu
    var = jnp.mean(mc * mc, axis=1, keepdims=True)
    m = jax.nn.gelu(mc / jnp.sqrt(var + 1e-5))
    z = jnp.dot(w2_ref[...], m, preferred_element_type=jnp.float32)
    z = z + b2_ref[...]
    z_ref[0] = z
    s = jnp.sum(z, axis=1, keepdims=True)
    s2 = jnp.sum(z * z, axis=1, keepdims=True)

    @pl.when(bidx == 0)
    def _():
        st_ref[...] = jnp.zeros_like(st_ref)

    st_ref[...] += jnp.concatenate([s, s2], axis=1)


def _mr_fc2(ycn, mx, wy, wd, mb, w2, b2):
    return pl.pallas_call(
        _mr_fc2_body,
        grid=(_B,),
        in_specs=[
            pl.BlockSpec((1, _C, _N), lambda b: (b, 0, 0)),
            pl.BlockSpec((1, _N, _C), lambda b: (b, 0, 0)),
            pl.BlockSpec((2 * _C, _C // 4), lambda b: (0, 0)),
            pl.BlockSpec((2 * _C, _C // 4), lambda b: (0, 0)),
            pl.BlockSpec((2 * _C, 1), lambda b: (0, 0)),
            pl.BlockSpec((_C, 2 * _C), lambda b: (0, 0)),
            pl.BlockSpec((_C, 1), lambda b: (0, 0)),
        ],
        out_specs=[
            pl.BlockSpec((1, _C, _N), lambda b: (b, 0, 0)),
            pl.BlockSpec((_C, 2), lambda b: (0, 0)),
        ],
        out_shape=[
            jax.ShapeDtypeStruct((_B, _C, _N), jnp.float32),
            jax.ShapeDtypeStruct((_C, 2), jnp.float32),
        ],
    )(ycn, mx, wy, wd, mb, w2, b2)


def _scconv_common(z_ref, st_ref, g2_ref, b2_ref, tmp_ref, gnw_ref, gnb_ref,
                   sq1_ref, sq2_ref, gwc_ref, gwcb_ref, pw1_ref, pw2_ref):
    mean = st_ref[:, 0:1] / _CNT
    var = st_ref[:, 1:2] / _CNT - mean * mean
    scale = g2_ref[...] / jnp.sqrt(var + 1e-5)
    shift = b2_ref[...] - mean * scale
    xin = z_ref[0] * scale + shift + tmp_ref[0]
    gs = []
    for g in range(4):
        xg = xin[g * 24:(g + 1) * 24]
        mg = jnp.mean(xg)
        xc = xg - mg
        sg = jnp.sqrt(jnp.mean(xc * xc))
        gs.append(xc / (sg + 1e-10))
    gn = jnp.concatenate(gs, axis=0) * gnw_ref[...] + gnb_ref[...]
    wg = gnw_ref[...] / jnp.sum(gnw_ref[...])
    rw = jax.nn.sigmoid(gn * wg)
    info = (rw >= 0.5).astype(jnp.float32)
    x1 = info * gn
    x2 = gn - x1
    yy = jnp.concatenate([x1[:48] + x2[48:], x1[48:] + x2[:48]], axis=0)
    up = jnp.dot(sq1_ref[...], yy[:48], preferred_element_type=jnp.float32)
    low = jnp.dot(sq2_ref[...], yy[48:], preferred_element_type=jnp.float32)

    li = lax.broadcasted_iota(jnp.int32, (1, _N), 1)
    hh = li // _W
    ww = li % _W
    top = jnp.zeros((48, _N), jnp.float32)
    bot = jnp.zeros((48, _N), jnp.float32)
    for t in range(9):
        dy, dx = t // 3 - 1, t % 3 - 1
        s = (dy * _W + dx) % _N
        rolled = up if s == 0 else jnp.concatenate([up[:, s:], up[:, :s]],
                                                   axis=1)
        valid = ((hh + dy >= 0) & (hh + dy < _H)
                 & (ww + dx >= 0) & (ww + dx < _W))
        sh = rolled * valid.astype(jnp.float32)
        wt = gwc_ref[t]
        top = top + jnp.dot(wt[0:48], sh[0:12],
                            preferred_element_type=jnp.float32)
        bot = bot + jnp.dot(wt[48:96], sh[12:24],
                            preferred_element_type=jnp.float32)
    y1 = jnp.concatenate([top, bot], axis=0)
    y1 = y1 + jnp.dot(pw1_ref[...], up, preferred_element_type=jnp.float32)
    y1 = y1 + gwcb_ref[...]
    y2 = jnp.concatenate(
        [jnp.dot(pw2_ref[...], low, preferred_element_type=jnp.float32), low],
        axis=0)
    out = jnp.concatenate([y1, y2], axis=0)
    mu = jnp.mean(out, axis=1, keepdims=True)
    a = jnp.exp(mu - jnp.max(mu))
    att = a / jnp.sum(a)
    out = out * att
    res = out[:96] + out[96:]
    rm = jnp.mean(res, axis=1, keepdims=True)
    rc = res - rm
    rv = jnp.mean(rc * rc, axis=1, keepdims=True)
    return rc / jnp.sqrt(rv + 1e-5)


def _scconv_mid_body(z_ref, st_ref, g2_ref, b2_ref, tmp_ref, gnw_ref, gnb_ref,
                     sq1_ref, sq2_ref, gwc_ref, gwcb_ref, pw1_ref, pw2_ref,
                     fw_ref, fb_ref, t2_ref, y2_ref, st2_ref):
    bidx = pl.program_id(0)
    rn = _scconv_common(z_ref, st_ref, g2_ref, b2_ref, tmp_ref, gnw_ref,
                        gnb_ref, sq1_ref, sq2_ref, gwc_ref, gwcb_ref,
                        pw1_ref, pw2_ref)
    t2 = jnp.maximum(rn, 0.0)
    t2_ref[0] = t2
    y2 = jnp.dot(fw_ref[...], t2, preferred_element_type=jnp.float32)
    y2 = y2 + fb_ref[...]
    y2_ref[0] = y2

    @pl.when(bidx == 0)
    def _():
        st2_ref[...] = jnp.zeros_like(st2_ref)

    acc = st2_ref[...]
    for nt in range(_N // 128):
        acc = acc + y2[:, nt * 128:(nt + 1) * 128]
    st2_ref[...] = acc


def _scconv_final_body(z_ref, st_ref, g2_ref, b2_ref, tmp_ref, gnw_ref,
                       gnb_ref, sq1_ref, sq2_ref, gwc_ref, gwcb_ref, pw1_ref,
                       pw2_ref, x0_ref, out_ref):
    rn = _scconv_common(z_ref, st_ref, g2_ref, b2_ref, tmp_ref, gnw_ref,
                        gnb_ref, sq1_ref, sq2_ref, gwc_ref, gwcb_ref,
                        pw1_ref, pw2_ref)
    out_ref[0] = x0_ref[0] + rn


_SC_IN_SPECS = [
    pl.BlockSpec((1, _C, _N), lambda b: (b, 0, 0)),
    pl.BlockSpec((_C, 2), lambda b: (0, 0)),
    pl.BlockSpec((_C, 1), lambda b: (0, 0)),
    pl.BlockSpec((_C, 1), lambda b: (0, 0)),
    pl.BlockSpec((1, _C, _N), lambda b: (b, 0, 0)),
    pl.BlockSpec((_C, 1), lambda b: (0, 0)),
    pl.BlockSpec((_C, 1), lambda b: (0, 0)),
    pl.BlockSpec((24, 48), lambda b: (0, 0)),
    pl.BlockSpec((24, 48), lambda b: (0, 0)),
    pl.BlockSpec((9, _C, 12), lambda b: (0, 0, 0)),
    pl.BlockSpec((_C, 1), lambda b: (0, 0)),
    pl.BlockSpec((_C, 24), lambda b: (0, 0)),
    pl.BlockSpec((_C - 24, 24), lambda b: (0, 0)),
]


def _scconv_mid(z, st, g2, b2, tmp, gnw, gnb, sq1, sq2, gwc, gwcb, pw1, pw2,
                fw, fb):
    return pl.pallas_call(
        _scconv_mid_body,
        grid=(_B,),
        in_specs=_SC_IN_SPECS + [
            pl.BlockSpec((_C, _C), lambda b: (0, 0)),
            pl.BlockSpec((_C, 1), lambda b: (0, 0)),
        ],
        out_specs=[
            pl.BlockSpec((1, _C, _N), lambda b: (b, 0, 0)),
            pl.BlockSpec((1, _C, _N), lambda b: (b, 0, 0)),
            pl.BlockSpec((_C, 128), lambda b: (0, 0)),
        ],
        out_shape=[
            jax.ShapeDtypeStruct((_B, _C, _N), jnp.float32),
            jax.ShapeDtypeStruct((_B, _C, _N), jnp.float32),
            jax.ShapeDtypeStruct((_C, 128), jnp.float32),
        ],
    )(z, st, g2, b2, tmp, gnw, gnb, sq1, sq2, gwc, gwcb, pw1, pw2, fw, fb)


def _scconv_final(z, st, g2, b2, tmp, gnw, gnb, sq1, sq2, gwc, gwcb, pw1, pw2,
                  x0):
    return pl.pallas_call(
        _scconv_final_body,
        grid=(_B,),
        in_specs=_SC_IN_SPECS + [
            pl.BlockSpec((1, _C, _N), lambda b: (b, 0, 0)),
        ],
        out_specs=pl.BlockSpec((1, _C, _N), lambda b: (b, 0, 0)),
        out_shape=jax.ShapeDtypeStruct((_B, _C, _N), jnp.float32),
    )(z, st, g2, b2, tmp, gnw, gnb, sq1, sq2, gwc, gwcb, pw1, pw2, x0)


def _col(v):
    return v.reshape(-1, 1)


def _split_mr(w):
    wr = w.reshape(4, 48, 48)
    wy = wr[:, :, 0::2].reshape(2 * _C, _C // 4)
    wd = wr[:, :, 1::2].reshape(2 * _C, _C // 4)
    return wy, wd


def _taps(gwc_w):
    return gwc_w.transpose(2, 3, 0, 1).reshape(9, _C, 12)


def _neighbor_major(idx):
    return jnp.transpose(idx, (2, 0, 1)).reshape(8 * _R // 128, 128)


def _grapher(x0, fc1_w, fc1_b, bn1_g, bn1_b, rel, mr_w, mr_b, fc2_w, fc2_b,
             y_raw=None, st=None):
    if y_raw is None:
        y_raw, st = _fc_stats(x0, fc1_w, _col(fc1_b))
    vv = _center_var(y_raw, st)
    ycn, ync, idx = _knn(y_raw, st, vv, _col(bn1_g), _col(bn1_b), rel)
    mx = _gather_max(ync.reshape(_R, 128), _neighbor_major(idx))
    wy, wd = _split_mr(mr_w)
    return _mr_fc2(ycn, mx.reshape(_B, _N, _C), wy, wd, _col(mr_b), fc2_w,
                   _col(fc2_b))


def kernel(x, g1_fc1_w, g1_fc1_b, g1_bn1_g, g1_bn1_b, g1_rel, g1_mr_w,
           g1_mr_b, g1_fc2_w, g1_fc2_b, g1_bn2_g, g1_bn2_b,
           g2_fc1_w, g2_fc1_b, g2_bn1_g, g2_bn1_b, g2_rel, g2_mr_w,
           g2_mr_b, g2_fc2_w, g2_fc2_b, g2_bn2_g, g2_bn2_b,
           s1_gn_w, s1_gn_b, s1_sq1_w, s1_sq2_w, s1_gwc_w, s1_gwc_b,
           s1_pwc1_w, s1_pwc2_w,
           s2_gn_w, s2_gn_b, s2_sq1_w, s2_sq2_w, s2_gwc_w, s2_gwc_b,
           s2_pwc1_w, s2_pwc2_w):
    x0 = x.reshape(_B, _C, _N)
    z1, st1b = _grapher(x0, g1_fc1_w, g1_fc1_b, g1_bn1_g, g1_bn1_b, g1_rel,
                        g1_mr_w, g1_mr_b, g1_fc2_w, g1_fc2_b)
    t2, y2_raw, st2 = _scconv_mid(
        z1, st1b, _col(g1_bn2_g), _col(g1_bn2_b), x0, _col(s1_gn_w),
        _col(s1_gn_b), s1_sq1_w, s1_sq2_w, _taps(s1_gwc_w), _col(s1_gwc_b),
        s1_pwc1_w, s1_pwc2_w, g2_fc1_w, _col(g2_fc1_b))
    z2, st2b = _grapher(None, None, None, g2_bn1_g, g2_bn1_b, g2_rel,
                        g2_mr_w, g2_mr_b, g2_fc2_w, g2_fc2_b,
                        y_raw=y2_raw, st=st2)
    out = _scconv_final(
        z2, st2b, _col(g2_bn2_g), _col(g2_bn2_b), t2, _col(s2_gn_w),
        _col(s2_gn_b), s2_sq1_w, s2_sq2_w, _taps(s2_gwc_w), _col(s2_gwc_b),
        s2_pwc1_w, s2_pwc2_w, x0)
    return out.reshape(_B, _C, _H, _W)
